# Initial kernel scaffold; baseline (speedup 1.0000x reference)
import jax
import jax.numpy as jnp
from jax.experimental import pallas as pl


def kernel(logits):
    raise NotImplementedError("write your pallas kernel here")


# speedup vs baseline: 116.0352x; 116.0352x over previous
import functools

import jax
import jax.numpy as jnp
from jax.experimental import pallas as pl
from jax.experimental.pallas import tpu as pltpu

_TEMPERATURE = 0.8
_TOP_P = 0.95
_V = 100000
_VPAD = 100096
_ROWS_PER_BLOCK = 8
_ONE_KEY = 0x3F800000


def _tc_body(x_ref, o_ref):
    x = x_ref[...] * (1.0 / _TEMPERATURE)
    m = jnp.max(x, axis=1, keepdims=True)
    w = jnp.exp(x - m)
    z = jnp.sum(w, axis=1, keepdims=True)
    target = _TOP_P * z

    def body(_, lohi):
        lo, hi = lohi
        mid = lo + (hi - lo) // 2
        midf = jax.lax.bitcast_convert_type(mid, jnp.float32)
        f = jnp.sum(jnp.where(w > midf, w, 0.0), axis=1, keepdims=True)
        le = f <= target
        return jnp.where(le, lo, mid), jnp.where(le, mid, hi)

    lo0 = jnp.zeros((_ROWS_PER_BLOCK, 1), jnp.int32)
    hi0 = jnp.full((_ROWS_PER_BLOCK, 1), _ONE_KEY, jnp.int32)
    _, hi = jax.lax.fori_loop(0, 31, body, (lo0, hi0))
    tau = jax.lax.bitcast_convert_type(hi, jnp.float32)
    kept = jnp.where(w >= tau, w, 0.0)
    wsum = jnp.sum(kept, axis=1, keepdims=True)
    o_ref[...] = kept / wsum


def _tc_nucleus(xp):
    b = xp.shape[0]
    grid = b // _ROWS_PER_BLOCK
    return pl.pallas_call(
        _tc_body,
        grid=(grid,),
        in_specs=[pl.BlockSpec((_ROWS_PER_BLOCK, _VPAD), lambda i: (i, 0))],
        out_specs=pl.BlockSpec((_ROWS_PER_BLOCK, _VPAD), lambda i: (i, 0)),
        out_shape=jax.ShapeDtypeStruct((b, _VPAD), jnp.float32),
    )(xp)


@jax.jit
def kernel(logits):
    xp = jnp.pad(logits, ((0, 0), (0, _VPAD - _V)),
                 constant_values=-jnp.inf)
    probs = _tc_nucleus(xp)
    return probs[:, :_V]
